# Initial kernel scaffold; baseline (speedup 1.0000x reference)
import jax
import jax.numpy as jnp
from jax.experimental import pallas as pl


def kernel(x, active_points, params):
    raise NotImplementedError("write your pallas kernel here")


# speedup vs baseline: 1.1664x; 1.1664x over previous
import jax
import jax.numpy as jnp
from jax import lax
from jax.experimental import pallas as pl
from jax.experimental.pallas import tpu as pltpu

IN_CH = 256
HID = 256
NCLS = 10
N = 224 * 224
BLK = 512


def _ln_t(h, g, b):
    m = jnp.mean(h, axis=0, keepdims=True)
    v = jnp.mean((h - m) ** 2, axis=0, keepdims=True)
    return (h - m) * jax.lax.rsqrt(v + 1e-5) * g + b


def _head_t(xb, W0T, g0, b0, W1T, g1, b1, WoutT):
    h = lax.dot_general(W0T, xb, (((1,), (1,)), ((), ())),
                        preferred_element_type=jnp.float32)
    h = jnp.maximum(_ln_t(h, g0, b0), 0.0)
    h = jnp.dot(W1T, h, preferred_element_type=jnp.float32)
    h = jnp.maximum(_ln_t(h, g1, b1), 0.0)
    return jnp.dot(WoutT, h, preferred_element_type=jnp.float32)


def _body(x_ref, mask_ref,
          cW0, cg0, cb0, cW1, cg1, cb1, cWo, cbo,
          rW0, rg0, rb0, rW1, rg1, rb1, rWo, rbo,
          dW0, dg0, db0, dW1, dg1, db1, dWo, dbo,
          fW0, fg0, fb0, fW1, fg1, fb1, fWo,
          out_cls, out_reg, out_dir, out_feat):
    xb = x_ref[...]
    m = mask_ref[...]
    o_cls = _head_t(xb, cW0[...], cg0[...], cb0[...], cW1[...], cg1[...],
                    cb1[...], cWo[...]) + cbo[...]
    o_reg = _head_t(xb, rW0[...], rg0[...], rb0[...], rW1[...], rg1[...],
                    rb1[...], rWo[...]) + rbo[...]
    o_dir = _head_t(xb, dW0[...], dg0[...], db0[...], dW1[...], dg1[...],
                    db1[...], dWo[...]) + dbo[...]
    o_feat = _head_t(xb, fW0[...], fg0[...], fb0[...], fW1[...], fg1[...],
                     fb1[...], fWo[...])
    out_cls[...] = o_cls * m
    out_reg[...] = o_reg * m
    out_dir[...] = o_dir * m
    out_feat[...] = o_feat * m


def _col(v):
    return jnp.reshape(v, (-1, 1))


def kernel(x, active_points, params):
    B, H, W, _ = x.shape
    n = B * H * W
    x2 = jnp.reshape(x, (n, IN_CH))
    mask = jnp.reshape(active_points, (1, n)).astype(jnp.float32)

    def head_args(p, final_bias):
        a = [p['W0'].T, _col(p['g0']), _col(p['b0']),
             p['W1'].T, _col(p['g1']), _col(p['b1']),
             p['Wout'].T]
        if final_bias:
            a.append(_col(p['bout']))
        return a

    weights = (head_args(params['cls'], True)
               + head_args(params['reg'], True)
               + head_args(params['dir'], True)
               + head_args(params['feat'], False))

    grid = (n // BLK,)
    row_spec = pl.BlockSpec((BLK, IN_CH), lambda j: (j, 0))
    mask_spec = pl.BlockSpec((1, BLK), lambda j: (0, j))

    def wspec(w):
        return pl.BlockSpec(w.shape, lambda j: (0, 0))

    out_shapes = (
        jax.ShapeDtypeStruct((NCLS, n), jnp.float32),
        jax.ShapeDtypeStruct((2, n), jnp.float32),
        jax.ShapeDtypeStruct((2, n), jnp.float32),
        jax.ShapeDtypeStruct((HID, n), jnp.float32),
    )
    out_specs = (
        pl.BlockSpec((NCLS, BLK), lambda j: (0, j)),
        pl.BlockSpec((2, BLK), lambda j: (0, j)),
        pl.BlockSpec((2, BLK), lambda j: (0, j)),
        pl.BlockSpec((HID, BLK), lambda j: (0, j)),
    )

    o_cls, o_reg, o_dir, o_feat = pl.pallas_call(
        _body,
        grid=grid,
        in_specs=[row_spec, mask_spec] + [wspec(w) for w in weights],
        out_specs=out_specs,
        out_shape=out_shapes,
        compiler_params=pltpu.CompilerParams(
            dimension_semantics=("arbitrary",)),
    )(x2, mask, *weights)

    return (jnp.reshape(o_cls, (B, NCLS, H, W)),
            jnp.reshape(o_reg, (B, 2, H, W)),
            jnp.reshape(o_dir, (B, 2, H, W)),
            jnp.reshape(o_feat, (B, HID, H, W)))
